# Initial kernel scaffold; baseline (speedup 1.0000x reference)
import jax
import jax.numpy as jnp
from jax.experimental import pallas as pl


def kernel(inputs, embedding_weight):
    raise NotImplementedError("write your pallas kernel here")


# speedup vs baseline: 1.2191x; 1.2191x over previous
import functools

import jax
import jax.numpy as jnp
from jax.experimental import pallas as pl

_NUM_E = 1024
_DIM = 256
_EPS = 1e-12
_COMMIT = 0.25


def _vq_body(x_ref, w_ref, loss_ref, q_ref, idx_ref, *, nbatch, t_len):
    b = pl.program_id(0)

    w = w_ref[...]
    wn = w / jnp.maximum(
        jnp.sqrt(jnp.sum(w * w, axis=1, keepdims=True)), _EPS)

    x = x_ref[0]
    xn = x / jnp.maximum(
        jnp.sqrt(jnp.sum(x * x, axis=0, keepdims=True)), _EPS)

    s = jax.lax.dot_general(wn, xn, (((1,), (0,)), ((), ())),
                            preferred_element_type=jnp.float32)
    d = 2.0 - 2.0 * s
    dmin = jnp.min(d, axis=0, keepdims=True)
    iota = jax.lax.broadcasted_iota(jnp.int32, d.shape, 0)
    idx = jnp.min(jnp.where(d == dmin, iota, _NUM_E), axis=0)
    idx_ref[...] = idx[None, None, :]

    onehot = (iota == idx[None, :]).astype(jnp.float32)
    q = jax.lax.dot_general(wn, onehot, (((0,), (0,)), ((), ())),
                            preferred_element_type=jnp.float32)
    q_ref[0] = q

    part = jnp.sum((q - xn) ** 2, keepdims=True).reshape(1, 1)

    @pl.when(b == 0)
    def _init():
        loss_ref[...] = jnp.zeros((1, 1), jnp.float32)

    denom = nbatch * t_len * _DIM
    loss_ref[...] += (1.0 + _COMMIT) * part / denom


def kernel(inputs, embedding_weight):
    nbatch, dim, t_len = inputs.shape
    body = functools.partial(_vq_body, nbatch=nbatch, t_len=t_len)
    loss2d, quantized, idx3d = pl.pallas_call(
        body,
        grid=(nbatch,),
        in_specs=[
            pl.BlockSpec((1, dim, t_len), lambda b: (b, 0, 0)),
            pl.BlockSpec((_NUM_E, dim), lambda b: (0, 0)),
        ],
        out_specs=[
            pl.BlockSpec((1, 1), lambda b: (0, 0)),
            pl.BlockSpec((1, dim, t_len), lambda b: (b, 0, 0)),
            pl.BlockSpec((1, 1, t_len), lambda b: (b, 0, 0)),
        ],
        out_shape=[
            jax.ShapeDtypeStruct((1, 1), jnp.float32),
            jax.ShapeDtypeStruct((nbatch, dim, t_len), jnp.float32),
            jax.ShapeDtypeStruct((nbatch, 1, t_len), jnp.int32),
        ],
    )(inputs, embedding_weight)
    loss = loss2d[0, 0]
    encoding_indices = idx3d.reshape(nbatch * t_len, 1)
    return (loss, quantized, encoding_indices, 0)


# speedup vs baseline: 1.2842x; 1.0534x over previous
import functools

import jax
import jax.numpy as jnp
from jax.experimental import pallas as pl
from jax.experimental.pallas import tpu as pltpu

_NUM_E = 1024
_DIM = 256
_EPS = 1e-12
_COMMIT = 0.25


def _vq_body(x_ref, nm_ref, w_ref, nw_ref, loss_ref, q_ref, idx_ref,
             wn_ref, *, nbatch, t_len, bb):
    b = pl.program_id(0)

    @pl.when(b == 0)
    def _prep():
        wn_ref[...] = w_ref[...] / nw_ref[...]
        loss_ref[...] = jnp.zeros((1, 1), jnp.float32)

    wn = wn_ref[...]
    part = jnp.zeros((1, 1), jnp.float32)

    for j in range(bb):
        xn = x_ref[j] / nm_ref[j]

        s = jax.lax.dot_general(wn, xn, (((1,), (0,)), ((), ())),
                                preferred_element_type=jnp.float32)
        d = 2.0 - 2.0 * s
        dmin = jnp.min(d, axis=0, keepdims=True)
        iota = jax.lax.broadcasted_iota(jnp.int32, d.shape, 0)
        idx = jnp.min(jnp.where(d == dmin, iota, _NUM_E), axis=0)
        idx_ref[j] = idx[None, :]

        onehot = (iota == idx[None, :]).astype(jnp.float32)
        q = jax.lax.dot_general(wn, onehot, (((0,), (0,)), ((), ())),
                                preferred_element_type=jnp.float32)
        q_ref[j] = q
        part = part + jnp.sum(dmin, keepdims=True).reshape(1, 1)

    denom = nbatch * t_len * _DIM
    loss_ref[...] += (1.0 + _COMMIT) * part / denom


def kernel(inputs, embedding_weight):
    nbatch, dim, t_len = inputs.shape
    bb = 2

    flat = jnp.transpose(inputs, (0, 2, 1)).reshape(-1, dim)
    nm = jnp.maximum(jnp.sqrt(jnp.sum(flat * flat, axis=1, keepdims=True)), _EPS)
    nmc = nm.reshape(nbatch, 1, t_len)
    w = embedding_weight
    nw = jnp.maximum(jnp.sqrt(jnp.sum(w * w, axis=1, keepdims=True)), _EPS)

    body = functools.partial(_vq_body, nbatch=nbatch, t_len=t_len, bb=bb)
    loss2d, quantized, idx3d = pl.pallas_call(
        body,
        grid=(nbatch // bb,),
        in_specs=[
            pl.BlockSpec((bb, dim, t_len), lambda b: (b, 0, 0)),
            pl.BlockSpec((bb, 1, t_len), lambda b: (b, 0, 0)),
            pl.BlockSpec((_NUM_E, dim), lambda b: (0, 0)),
            pl.BlockSpec((_NUM_E, 1), lambda b: (0, 0)),
        ],
        out_specs=[
            pl.BlockSpec((1, 1), lambda b: (0, 0)),
            pl.BlockSpec((bb, dim, t_len), lambda b: (b, 0, 0)),
            pl.BlockSpec((bb, 1, t_len), lambda b: (b, 0, 0)),
        ],
        out_shape=[
            jax.ShapeDtypeStruct((1, 1), jnp.float32),
            jax.ShapeDtypeStruct((nbatch, dim, t_len), jnp.float32),
            jax.ShapeDtypeStruct((nbatch, 1, t_len), jnp.int32),
        ],
        scratch_shapes=[pltpu.VMEM((_NUM_E, _DIM), jnp.float32)],
    )(inputs, nmc, w, nw)
    loss = loss2d[0, 0]
    encoding_indices = idx3d.reshape(nbatch * t_len, 1)
    return (loss, quantized, encoding_indices, 0)
